# baseline (speedup 1.0000x reference)
import functools

import jax
import jax.numpy as jnp
from jax import lax
from jax.experimental import pallas as pl
from jax.experimental.pallas import tpu as pltpu
from jax.experimental.pallas import tpu_sc as plsc

_VOCAB = 1000000
_EMBED = 32
_BATCH = 4096
_SEQ = 200

_BLK1 = 131072


def _proj_body(x_ref, w_ref, o_ref):
    o_ref[...] = jnp.sum(x_ref[...] * w_ref[:, 0:1], axis=0)


def _project_table(emb_t, w_col):
    return pl.pallas_call(
        _proj_body,
        grid=(pl.cdiv(_VOCAB, _BLK1),),
        in_specs=[
            pl.BlockSpec((_EMBED, _BLK1), lambda i: (0, i)),
            pl.BlockSpec((_EMBED, 128), lambda i: (0, 0)),
        ],
        out_specs=pl.BlockSpec((_BLK1,), lambda i: (i,)),
        out_shape=jax.ShapeDtypeStruct((_VOCAB,), jnp.float32),
    )(emb_t, w_col)


_NW = 32
_RPT = _BATCH // _NW
_IPT = _RPT * _SEQ
_NGRP = _RPT // 16


@functools.partial(
    pl.kernel,
    mesh=plsc.VectorSubcoreMesh(core_axis_name="c", subcore_axis_name="s"),
    out_type=jax.ShapeDtypeStruct((_BATCH,), jnp.float32),
    scratch_types=[
        pltpu.VMEM((_SEQ, _RPT), jnp.int32),
        pltpu.VMEM((_IPT,), jnp.float32),
        pltpu.VMEM((16,), jnp.float32),
        pltpu.VMEM((_RPT,), jnp.float32),
        pltpu.SemaphoreType.DMA,
    ],
)
def _sc_pool(ids_hbm, v_hbm, bias_hbm, out_hbm, idx_v, vals_v, bias_v,
             out_v, sem):
    wid = lax.axis_index("s") * 2 + lax.axis_index("c")
    base = pl.multiple_of(wid * _RPT, _RPT)
    pltpu.sync_copy(ids_hbm.at[:, pl.ds(base, _RPT)], idx_v)
    pltpu.sync_copy(bias_hbm, bias_v)

    def _fire(j, carry):
        off = pl.multiple_of(j * _RPT, _RPT)
        pltpu.async_copy(
            v_hbm.at[idx_v.at[j]],
            vals_v.at[pl.ds(off, _RPT)],
            sem,
        )
        return carry

    lax.fori_loop(0, _SEQ, _fire, 0, unroll=4)
    pltpu.make_async_copy(v_hbm.at[pl.ds(0, _IPT)], vals_v, sem).wait()

    bias = bias_v[...]

    def _accum(j, accs):
        row = pl.multiple_of(j * _RPT, _RPT)
        return tuple(
            accs[g] + vals_v[pl.ds(row + g * 16, 16)]
            for g in range(_NGRP)
        )

    zeros = jnp.zeros((16,), jnp.float32)
    accs = lax.fori_loop(0, _SEQ, _accum, tuple(zeros for _ in range(_NGRP)),
                         unroll=2)
    for g in range(_NGRP):
        out_v[pl.ds(g * 16, 16)] = accs[g] + bias

    pltpu.sync_copy(out_v, out_hbm.at[pl.ds(base, _RPT)])


def kernel(input_ids, embedding, fc_w, fc_b):
    ids_t = jnp.transpose(input_ids.astype(jnp.int32))
    emb_t = jnp.transpose(embedding)
    w = fc_w.astype(jnp.float32).reshape(_EMBED) * (1.0 / _SEQ)
    w_col = jnp.broadcast_to(w[:, None], (_EMBED, 128))
    v = _project_table(emb_t, w_col)
    bias_vec = jnp.broadcast_to(fc_b.astype(jnp.float32).reshape(1), (16,))
    out = _sc_pool(ids_t, v, bias_vec)
    return out.reshape(_BATCH, 1)


# speedup vs baseline: 1.0042x; 1.0042x over previous
import functools

import jax
import jax.numpy as jnp
from jax import lax
from jax.experimental import pallas as pl
from jax.experimental.pallas import tpu as pltpu
from jax.experimental.pallas import tpu_sc as plsc

_VOCAB = 1000000
_EMBED = 32
_BATCH = 4096
_SEQ = 200

_BLK1 = 65536


def _proj_body(x_ref, w_ref, o_ref):
    o_ref[...] = jnp.sum(x_ref[...] * w_ref[:, 0:1], axis=0)


def _project_table(emb_t, w_col):
    return pl.pallas_call(
        _proj_body,
        grid=(pl.cdiv(_VOCAB, _BLK1),),
        in_specs=[
            pl.BlockSpec((_EMBED, _BLK1), lambda i: (0, i)),
            pl.BlockSpec((_EMBED, 128), lambda i: (0, 0)),
        ],
        out_specs=pl.BlockSpec((_BLK1,), lambda i: (i,)),
        out_shape=jax.ShapeDtypeStruct((_VOCAB,), jnp.float32),
    )(emb_t, w_col)


_NW = 32
_RPT = _BATCH // _NW
_IPT = _RPT * _SEQ
_NGRP = _RPT // 16


@functools.partial(
    pl.kernel,
    mesh=plsc.VectorSubcoreMesh(core_axis_name="c", subcore_axis_name="s"),
    out_type=jax.ShapeDtypeStruct((_BATCH,), jnp.float32),
    scratch_types=[
        pltpu.VMEM((_SEQ, _RPT), jnp.int32),
        pltpu.VMEM((_IPT,), jnp.float32),
        pltpu.VMEM((16,), jnp.float32),
        pltpu.VMEM((_RPT,), jnp.float32),
        pltpu.SemaphoreType.DMA,
    ],
)
def _sc_pool(ids_hbm, v_hbm, bias_hbm, out_hbm, idx_v, vals_v, bias_v,
             out_v, sem):
    wid = lax.axis_index("s") * 2 + lax.axis_index("c")
    base = pl.multiple_of(wid * _RPT, _RPT)
    pltpu.sync_copy(ids_hbm.at[:, pl.ds(base, _RPT)], idx_v)
    pltpu.sync_copy(bias_hbm, bias_v)

    def _fire(j, carry):
        off = pl.multiple_of(j * _RPT, _RPT)
        pltpu.async_copy(
            v_hbm.at[idx_v.at[j]],
            vals_v.at[pl.ds(off, _RPT)],
            sem,
        )
        return carry

    lax.fori_loop(0, _SEQ, _fire, 0, unroll=4)
    pltpu.make_async_copy(v_hbm.at[pl.ds(0, _IPT)], vals_v, sem).wait()

    bias = bias_v[...]

    def _accum(j, accs):
        row = pl.multiple_of(j * _RPT, _RPT)
        return tuple(
            accs[g] + vals_v[pl.ds(row + g * 16, 16)]
            for g in range(_NGRP)
        )

    zeros = jnp.zeros((16,), jnp.float32)
    accs = lax.fori_loop(0, _SEQ, _accum, tuple(zeros for _ in range(_NGRP)),
                         unroll=2)
    for g in range(_NGRP):
        out_v[pl.ds(g * 16, 16)] = accs[g] + bias

    pltpu.sync_copy(out_v, out_hbm.at[pl.ds(base, _RPT)])


def kernel(input_ids, embedding, fc_w, fc_b):
    ids_t = jnp.transpose(input_ids.astype(jnp.int32))
    emb_t = jnp.transpose(embedding)
    w = fc_w.astype(jnp.float32).reshape(_EMBED) * (1.0 / _SEQ)
    w_col = jnp.broadcast_to(w[:, None], (_EMBED, 128))
    v = _project_table(emb_t, w_col)
    bias_vec = jnp.broadcast_to(fc_b.astype(jnp.float32).reshape(1), (16,))
    out = _sc_pool(ids_t, v, bias_vec)
    return out.reshape(_BATCH, 1)
